# baseline (speedup 1.0000x reference)
import functools

import jax
import jax.numpy as jnp
import numpy as np
from jax import lax
from jax.experimental import pallas as pl
from jax.experimental.pallas import tpu as pltpu
from jax.experimental.pallas import tpu_sc as plsc

VOCAB = 100000
D = 128
BATCH = 4096
HIST = 50

_info = plsc.get_sparse_core_info()
NC = _info.num_cores
NS = _info.num_subcores
NW = NC * NS
LANES = 16

SPLIT = 2
BSPLIT = BATCH // SPLIT
NH = BSPLIT * HIST

BPW = NH // NW
CHUNK = 128
NCHUNK = BPW // CHUNK

BB = 16

SCALE = float(np.sqrt(np.float32(D)))


def _gather_body(idx_hbm, table_hbm, out_hbm,
                 idx_v, rin0, rin1, rout0, rout1,
                 gs0, gs1, os0, os1):
    wid = lax.axis_index("s") * NC + lax.axis_index("c")
    base = wid * BPW

    pltpu.sync_copy(idx_hbm.at[wid], idx_v)

    bufs = ((rin0, gs0, rout0, os0), (rin1, gs1, rout1, os1))

    def start_gather(c, rin, gsem):
        pltpu.async_copy(table_hbm.at[idx_v.at[c]], rin, gsem)

    def wait_gather(c, rin, gsem):
        pltpu.make_async_copy(table_hbm.at[idx_v.at[c]], rin, gsem).wait()

    def start_out(c, rout, osem):
        pltpu.async_copy(rout, out_hbm.at[pl.ds(base + c * CHUNK, CHUNK)], osem)

    def wait_out(c, rout, osem):
        pltpu.make_async_copy(
            rout, out_hbm.at[pl.ds(base + c * CHUNK, CHUNK)], osem).wait()

    def scale_chunk(rin, rout):
        def row_body(r, _):
            for j in range(D // LANES):
                sl = pl.ds(j * LANES, LANES)
                rout[r, sl] = rin[r, sl] * SCALE
            return 0

        lax.fori_loop(0, CHUNK, row_body, 0)

    start_gather(0, rin0, gs0)
    start_gather(1, rin1, gs1)
    for b in range(2):
        rin, gsem, rout, osem = bufs[b]
        wait_gather(b, rin, gsem)
        scale_chunk(rin, rout)
        start_out(b, rout, osem)
        start_gather(b + 2, rin, gsem)

    CE = 2 * ((NCHUNK - 2) // 2)

    def steady(g, _):
        for b in range(2):
            c = 2 * g + b
            rin, gsem, rout, osem = bufs[b]
            wait_gather(c, rin, gsem)
            wait_out(c - 2, rout, osem)
            scale_chunk(rin, rout)
            start_out(c, rout, osem)
            start_gather(c + 2, rin, gsem)
        return 0

    lax.fori_loop(1, CE // 2, steady, 0)

    for c in range(CE, NCHUNK):
        rin, gsem, rout, osem = bufs[c % 2]
        wait_gather(c, rin, gsem)
        wait_out(c - 2, rout, osem)
        scale_chunk(rin, rout)
        start_out(c, rout, osem)
        if c + 2 < NCHUNK:
            start_gather(c + 2, rin, gsem)
    for c in range(NCHUNK - 2, NCHUNK):
        rin, gsem, rout, osem = bufs[c % 2]
        wait_out(c, rout, osem)


def _sc_gather_half(idx, table):
    call = functools.partial(
        pl.kernel,
        mesh=plsc.VectorSubcoreMesh(core_axis_name="c", subcore_axis_name="s"),
        out_type=jax.ShapeDtypeStruct((NH, D), jnp.float32),
        scratch_types=[
            pltpu.VMEM((NCHUNK, CHUNK), jnp.int32),
            pltpu.VMEM((CHUNK, D), jnp.float32),
            pltpu.VMEM((CHUNK, D), jnp.float32),
            pltpu.VMEM((CHUNK, D), jnp.float32),
            pltpu.VMEM((CHUNK, D), jnp.float32),
            pltpu.SemaphoreType.DMA,
            pltpu.SemaphoreType.DMA,
            pltpu.SemaphoreType.DMA,
            pltpu.SemaphoreType.DMA,
        ],
    )(_gather_body)
    return call(idx, table)


def _relayout_body(flat_ref, o_ref):
    for b in range(BB):
        o_ref[b] = flat_ref[pl.ds(b * HIST, HIST), :]


def _tc_relayout(flat, half_idx, carry=None):
    grid = (BSPLIT // BB,)
    off = half_idx * (BSPLIT // BB)
    in_specs = [pl.BlockSpec((BB * HIST, D), lambda i: (i, 0))]
    operands = [flat]
    aliases = {}
    if carry is not None:
        in_specs.append(pl.BlockSpec(memory_space=pl.ANY))
        operands.append(carry)
        aliases = {1: 0}
    body = _relayout_body if carry is None else (
        lambda flat_ref, carry_ref, o_ref: _relayout_body(flat_ref, o_ref))
    return pl.pallas_call(
        body,
        grid=grid,
        in_specs=in_specs,
        out_specs=pl.BlockSpec((BB, HIST, D), lambda i, off=off: (i + off, 0, 0)),
        out_shape=jax.ShapeDtypeStruct((BATCH, HIST, D), jnp.float32),
        input_output_aliases=aliases,
    )(*operands)


@jax.jit
def _embed(x, table):
    idx = x.reshape(SPLIT, NW, NCHUNK, CHUNK)
    flat0 = _sc_gather_half(idx[0], table)
    flat1 = _sc_gather_half(idx[1], table)
    out = _tc_relayout(flat0, 0)
    out = _tc_relayout(flat1, 1, carry=out)
    return out


def kernel(x, input_embedding_table):
    return _embed(x.astype(jnp.int32), input_embedding_table)


# speedup vs baseline: 1.0836x; 1.0836x over previous
import functools

import jax
import jax.numpy as jnp
import numpy as np
from jax import lax
from jax.experimental import pallas as pl
from jax.experimental.pallas import tpu as pltpu
from jax.experimental.pallas import tpu_sc as plsc

VOCAB = 100000
D = 128
BATCH = 4096
HIST = 50

_info = plsc.get_sparse_core_info()
NC = _info.num_cores
NS = _info.num_subcores
NW = NC * NS
LANES = 16

BPW = BATCH // NW
BBLK = 8
NBLK = BPW // BBLK
GPB = 4
GB = BBLK // GPB
CIDX = GB * HIST
CSTR = 104
BIDX = GPB * CSTR

SCALE = float(np.sqrt(np.float32(D)))


def _gather_body(idx_hbm, table_hbm, out_hbm,
                 ix0, ix1, rn0, rn1, ro0, ro1,
                 is0, is1, gs0, gs1, os0, os1):
    wid = lax.axis_index("s") * NC + lax.axis_index("c")
    base_b = wid * BPW

    ixs = ((ix0, is0), (ix1, is1))
    rns = ((rn0, gs0), (rn1, gs1))
    ros = ((ro0, os0), (ro1, os1))

    def start_idx(j, par):
        ix, sem = ixs[par]
        pltpu.async_copy(idx_hbm.at[wid, j], ix, sem)

    def wait_idx(j, par):
        ix, sem = ixs[par]
        pltpu.make_async_copy(idx_hbm.at[wid, j], ix, sem).wait()

    def start_gather(par, g, gi):
        ix, _ = ixs[par]
        rn, sem = rns[gi % 2]
        pltpu.async_copy(table_hbm.at[ix.at[pl.ds(g * CSTR, CIDX)]], rn, sem)

    def wait_gather(par, g, gi):
        ix, _ = ixs[par]
        rn, sem = rns[gi % 2]
        pltpu.make_async_copy(
            table_hbm.at[ix.at[pl.ds(g * CSTR, CIDX)]], rn, sem).wait()

    def out_slice(j):
        return out_hbm.at[:, pl.ds(base_b + j * BBLK, BBLK), :]

    def start_out(j, par):
        ro, sem = ros[par]
        pltpu.async_copy(ro, out_slice(j), sem)

    def wait_out(j, par):
        ro, sem = ros[par]
        pltpu.make_async_copy(ro, out_slice(j), sem).wait()

    def scale_gather(par, g, gi):
        rn, _ = rns[gi % 2]
        ro, _ = ros[par]

        def row_body(h, _):
            for b in range(GB):
                for q in range(D // LANES):
                    sl = pl.ds(q * LANES, LANES)
                    ro[h, g * GB + b, sl] = rn[h * GB + b, sl] * SCALE
            return 0

        lax.fori_loop(0, HIST, row_body, 0)

    def process_block(j, par, first, last):
        if not first:
            wait_out(j - 2, par)
        wait_idx(j, par)
        if not last:
            start_idx(j + 2, par)
        start_gather(par, 0, 0)
        start_gather(par, 1, 1)
        for g in range(GPB):
            wait_gather(par, g, g)
            scale_gather(par, g, g)
            if g + 2 < GPB:
                start_gather(par, g + 2, g + 2)
        start_out(j, par)

    start_idx(0, 0)
    start_idx(1, 1)
    process_block(0, 0, True, False)
    process_block(1, 1, True, False)

    def steady(jj, _):
        for p in range(2):
            j = 2 * jj + p
            process_block(j, p, False, False)
        return 0

    lax.fori_loop(1, NBLK // 2 - 1, steady, 0)

    process_block(NBLK - 2, 0, False, True)
    process_block(NBLK - 1, 1, False, True)
    wait_out(NBLK - 2, 0)
    wait_out(NBLK - 1, 1)


@jax.jit
def _embed(x, table):
    idx = (x.reshape(NW, NBLK, GPB, GB, HIST)
             .transpose(0, 1, 2, 4, 3)
             .reshape(NW, NBLK, GPB, CIDX))
    idx = jnp.pad(idx, ((0, 0), (0, 0), (0, 0), (0, CSTR - CIDX)))
    idx = idx.reshape(NW, NBLK, BIDX)
    call = functools.partial(
        pl.kernel,
        mesh=plsc.VectorSubcoreMesh(core_axis_name="c", subcore_axis_name="s"),
        out_type=jax.ShapeDtypeStruct((HIST, BATCH, D), jnp.float32),
        scratch_types=[
            pltpu.VMEM((BIDX,), jnp.int32),
            pltpu.VMEM((BIDX,), jnp.int32),
            pltpu.VMEM((CIDX, D), jnp.float32),
            pltpu.VMEM((CIDX, D), jnp.float32),
            pltpu.VMEM((HIST, BBLK, D), jnp.float32),
            pltpu.VMEM((HIST, BBLK, D), jnp.float32),
            pltpu.SemaphoreType.DMA,
            pltpu.SemaphoreType.DMA,
            pltpu.SemaphoreType.DMA,
            pltpu.SemaphoreType.DMA,
            pltpu.SemaphoreType.DMA,
            pltpu.SemaphoreType.DMA,
        ],
    )(_gather_body)
    hbd = call(idx, table)
    return jnp.transpose(hbd, (1, 0, 2))


def kernel(x, input_embedding_table):
    return _embed(x.astype(jnp.int32), input_embedding_table)


# speedup vs baseline: 1.0971x; 1.0124x over previous
import functools

import jax
import jax.numpy as jnp
import numpy as np
from jax import lax
from jax.experimental import pallas as pl
from jax.experimental.pallas import tpu as pltpu
from jax.experimental.pallas import tpu_sc as plsc

VOCAB = 100000
D = 128
BATCH = 4096
HIST = 50

_info = plsc.get_sparse_core_info()
NC = _info.num_cores
NS = _info.num_subcores
NW = NC * NS
LANES = 16

BPW = BATCH // NW
BBLK = 8
NBLK = BPW // BBLK
GPB = 4
GB = BBLK // GPB
CIDX = GB * HIST

SCALE = float(np.sqrt(np.float32(D)))


def _gather_body(idx_hbm, table_hbm, out_hbm,
                 ix0, ix1, rn0, rn1, ro0, ro1,
                 is0, is1, gs0, gs1, os0, os1):
    wid = lax.axis_index("s") * NC + lax.axis_index("c")
    base_b = wid * BPW

    ixs = ((ix0, is0), (ix1, is1))
    rns = ((rn0, gs0), (rn1, gs1))
    ros = ((ro0, os0), (ro1, os1))

    def start_idx(j, par):
        ix, sem = ixs[par]
        pltpu.async_copy(idx_hbm.at[wid, j], ix, sem)

    def wait_idx(j, par):
        ix, sem = ixs[par]
        pltpu.make_async_copy(idx_hbm.at[wid, j], ix, sem).wait()

    def start_gather(par, g, gi):
        ix, _ = ixs[par]
        rn, sem = rns[gi % 2]
        pltpu.async_copy(table_hbm.at[ix.at[g]], rn, sem)

    def wait_gather(par, g, gi):
        ix, _ = ixs[par]
        rn, sem = rns[gi % 2]
        pltpu.make_async_copy(table_hbm.at[ix.at[g]], rn, sem).wait()

    def out_slice(j):
        return out_hbm.at[:, pl.ds(base_b + j * BBLK, BBLK), :]

    def start_out(j, par):
        ro, sem = ros[par]
        pltpu.async_copy(ro, out_slice(j), sem)

    def wait_out(j, par):
        ro, sem = ros[par]
        pltpu.make_async_copy(ro, out_slice(j), sem).wait()

    def scale_gather(par, g, gi):
        rn, _ = rns[gi % 2]
        ro, _ = ros[par]

        def row_body(h, _):
            for b in range(GB):
                for q in range(D // LANES):
                    sl = pl.ds(q * LANES, LANES)
                    ro[h, g * GB + b, sl] = rn[h * GB + b, sl] * SCALE
            return 0

        lax.fori_loop(0, HIST, row_body, 0)

    def process_block(j, par, first, last):
        if not first:
            wait_out(j - 2, par)
        wait_idx(j, par)
        start_gather(par, 0, 0)
        start_gather(par, 1, 1)
        for g in range(GPB):
            wait_gather(par, g, g)
            scale_gather(par, g, g)
            if g + 2 < GPB:
                start_gather(par, g + 2, g + 2)
        if not last:
            start_idx(j + 2, par)
        start_out(j, par)

    start_idx(0, 0)
    start_idx(1, 1)
    process_block(0, 0, True, False)
    process_block(1, 1, True, False)

    def steady(jj, _):
        for p in range(2):
            j = 2 * jj + p
            process_block(j, p, False, False)
        return 0

    lax.fori_loop(1, NBLK // 2 - 1, steady, 0)

    process_block(NBLK - 2, 0, False, True)
    process_block(NBLK - 1, 1, False, True)
    wait_out(NBLK - 2, 0)
    wait_out(NBLK - 1, 1)


@jax.jit
def _embed(x, table):
    idx = (x.reshape(NW, NBLK, GPB, GB, HIST)
             .transpose(0, 1, 2, 4, 3)
             .reshape(NW, NBLK, GPB, CIDX))
    call = functools.partial(
        pl.kernel,
        mesh=plsc.VectorSubcoreMesh(core_axis_name="c", subcore_axis_name="s"),
        out_type=jax.ShapeDtypeStruct((HIST, BATCH, D), jnp.float32),
        scratch_types=[
            pltpu.VMEM((GPB, CIDX), jnp.int32),
            pltpu.VMEM((GPB, CIDX), jnp.int32),
            pltpu.VMEM((CIDX, D), jnp.float32),
            pltpu.VMEM((CIDX, D), jnp.float32),
            pltpu.VMEM((HIST, BBLK, D), jnp.float32),
            pltpu.VMEM((HIST, BBLK, D), jnp.float32),
            pltpu.SemaphoreType.DMA,
            pltpu.SemaphoreType.DMA,
            pltpu.SemaphoreType.DMA,
            pltpu.SemaphoreType.DMA,
            pltpu.SemaphoreType.DMA,
            pltpu.SemaphoreType.DMA,
        ],
    )(_gather_body)
    hbd = call(idx, table)
    return jnp.transpose(hbd, (1, 0, 2))


def kernel(x, input_embedding_table):
    return _embed(x.astype(jnp.int32), input_embedding_table)


# speedup vs baseline: 3.5193x; 3.2078x over previous
import functools

import jax
import jax.numpy as jnp
import numpy as np
from jax import lax
from jax.experimental import pallas as pl
from jax.experimental.pallas import tpu as pltpu
from jax.experimental.pallas import tpu_sc as plsc

VOCAB = 100000
D = 128
BATCH = 4096
HIST = 50

_info = plsc.get_sparse_core_info()
NC = _info.num_cores
NS = _info.num_subcores
NW = NC * NS
LANES = 16

BPW = BATCH // NW
NBUF = 4

SCALE = float(np.sqrt(np.float32(D)))


def _gather_body(idx_hbm, table_hbm, out_hbm,
                 ix, rn0, rn1, rn2, rn3, s0, s1, s2, s3):
    wid = lax.axis_index("s") * NC + lax.axis_index("c")
    base_b = wid * BPW

    bufs = ((rn0, s0), (rn1, s1), (rn2, s2), (rn3, s3))

    pltpu.sync_copy(idx_hbm.at[wid], ix)

    def start_gather(c, par):
        rn, sem = bufs[par]
        pltpu.async_copy(table_hbm.at[ix.at[c]], rn, sem)

    def wait_gather(c, par):
        rn, sem = bufs[par]
        pltpu.make_async_copy(table_hbm.at[ix.at[c]], rn, sem).wait()

    def out_slice(c):
        return out_hbm.at[c, pl.ds(base_b, BPW), :]

    def start_out(c, par):
        rn, sem = bufs[par]
        pltpu.async_copy(rn, out_slice(c), sem)

    def wait_out(c, par):
        rn, sem = bufs[par]
        pltpu.make_async_copy(rn, out_slice(c), sem).wait()

    def scale_chunk(par):
        rn, _ = bufs[par]

        def row_body(r, _):
            for q in range(D // LANES):
                sl = pl.ds(q * LANES, LANES)
                rn[r, sl] = rn[r, sl] * SCALE
            return 0

        lax.fori_loop(0, BPW, row_body, 0)

    def process(c, par, *, drain, issue):
        wait_gather(c, par)
        scale_chunk(par)
        start_out(c, par)
        if drain:
            wait_out(c - 1, (par + 3) % NBUF)
        if issue:
            start_gather(c + NBUF - 1, (par + 3) % NBUF)

    for c in range(NBUF - 1):
        start_gather(c, c)

    process(0, 0, drain=False, issue=True)
    for c in range(1, NBUF):
        process(c, c % NBUF, drain=True, issue=True)

    STEADY_ITERS = (HIST - 2 * (NBUF - 1)) // NBUF

    def steady(g, _):
        for p in range(NBUF):
            c = NBUF * g + p
            process(c, p, drain=True, issue=True)
        return 0

    lax.fori_loop(1, STEADY_ITERS, steady, 0)

    for c in range(NBUF * STEADY_ITERS, HIST):
        process(c, c % NBUF, drain=True, issue=(c + NBUF - 1 < HIST))

    wait_out(HIST - 1, (HIST - 1) % NBUF)


@jax.jit
def _embed(x, table):
    idx = x.reshape(NW, BPW, HIST).transpose(0, 2, 1)
    call = functools.partial(
        pl.kernel,
        mesh=plsc.VectorSubcoreMesh(core_axis_name="c", subcore_axis_name="s"),
        out_type=jax.ShapeDtypeStruct((HIST, BATCH, D), jnp.float32),
        scratch_types=[
            pltpu.VMEM((HIST, BPW), jnp.int32),
            pltpu.VMEM((BPW, D), jnp.float32),
            pltpu.VMEM((BPW, D), jnp.float32),
            pltpu.VMEM((BPW, D), jnp.float32),
            pltpu.VMEM((BPW, D), jnp.float32),
            pltpu.SemaphoreType.DMA,
            pltpu.SemaphoreType.DMA,
            pltpu.SemaphoreType.DMA,
            pltpu.SemaphoreType.DMA,
        ],
    )(_gather_body)
    hbd = call(idx, table)
    return jnp.transpose(hbd, (1, 0, 2))


def kernel(x, input_embedding_table):
    return _embed(x.astype(jnp.int32), input_embedding_table)
